# Initial kernel scaffold; baseline (speedup 1.0000x reference)
import jax
import jax.numpy as jnp
from jax.experimental import pallas as pl


def kernel(global_node_table, node_table, edge_table, hist_window_params, future_window_params, h_ratio, f_ratio, h_window, f_window, seed_nodes, relation_batch, glob_edge_index, glob_rel, h_edge_index, h_rel, f_edge_index, f_rel, neighbor_batch_size):
    raise NotImplementedError("write your pallas kernel here")


# speedup vs baseline: 1.0008x; 1.0008x over previous
import jax
import jax.numpy as jnp
from jax.experimental import pallas as pl

ENTITY_NUM = 50000
TIME_NUM = 4
N_TEMPORAL = ENTITY_NUM * TIME_NUM
BASE_WINDOW = 10.0
LAMBDA_STATIC = 0.5


def _gcn_layer(x, src, dst, rel, rel_emb, n_nodes, window=None):
    ones = jnp.ones(src.shape[0], dtype=x.dtype)
    out_deg = jax.ops.segment_sum(ones, src, num_segments=n_nodes)
    in_deg = jax.ops.segment_sum(ones, dst, num_segments=n_nodes)
    out_sqrt = jnp.sqrt(jnp.maximum(out_deg, 1.0))
    in_sqrt = jnp.sqrt(jnp.maximum(in_deg, 1.0))
    msg = x[src] * rel_emb[rel] / out_sqrt[src][:, None]
    if window is not None:
        msg = msg * window[:, None]
    agg = jax.ops.segment_sum(msg, dst, num_segments=n_nodes)
    return agg / in_sqrt[:, None]


def _two_layer(x, src, dst, rel, rel_emb, n_nodes, window=None):
    l1 = _gcn_layer(x, src, dst, rel, rel_emb, n_nodes, window) + x
    l2 = _gcn_layer(l1, src, dst, rel, rel_emb, n_nodes, window) + x
    return l2


def _dynamic_window(window_size, rel, window_params):
    w = jnp.clip(window_params, 0.0, 1.0)[rel, 0]
    return 1.0 / (1.0 + jnp.exp(window_size - BASE_WINDOW * w))


def _combine_body(h_ref, f_ref, g_ref, hr_ref, fr_ref, o_ref):
    o_ref[...] = (hr_ref[0] * h_ref[...] + fr_ref[0] * f_ref[...]
                  + LAMBDA_STATIC * g_ref[...])


def kernel(global_node_table, node_table, edge_table, hist_window_params,
           future_window_params, h_ratio, f_ratio, h_window, f_window,
           seed_nodes, relation_batch, glob_edge_index, glob_rel,
           h_edge_index, h_rel, f_edge_index, f_rel, neighbor_batch_size):
    glob_out = _two_layer(global_node_table, glob_edge_index[0], glob_edge_index[1],
                          glob_rel, edge_table, ENTITY_NUM, None)
    dw_h = _dynamic_window(h_window, h_rel, hist_window_params)
    h_out = _two_layer(node_table, h_edge_index[0], h_edge_index[1],
                       h_rel, edge_table, N_TEMPORAL, dw_h)
    dw_f = _dynamic_window(f_window, f_rel, future_window_params)
    f_out = _two_layer(node_table, f_edge_index[0], f_edge_index[1],
                       f_rel, edge_table, N_TEMPORAL, dw_f)
    orig = seed_nodes // TIME_NUM
    hs = h_out[seed_nodes]
    fs = f_out[seed_nodes]
    gs = glob_out[orig]
    out = pl.pallas_call(
        _combine_body,
        out_shape=jax.ShapeDtypeStruct(hs.shape, hs.dtype),
    )(hs, fs, gs, h_ratio, f_ratio)
    return out


# speedup vs baseline: 1.1697x; 1.1687x over previous
import functools

import jax
import jax.numpy as jnp
from jax import lax
from jax.experimental import pallas as pl
from jax.experimental.pallas import tpu as pltpu
from jax.experimental.pallas import tpu_sc as plsc

ENTITY_NUM = 50000
TIME_NUM = 4
N_TEMPORAL = ENTITY_NUM * TIME_NUM
EMB_DIM = 128
BASE_WINDOW = 10.0
LAMBDA_STATIC = 0.5
E = 512000
B = 1024

NS = 16
NC = 2
EPT = E // (NS * NC)
CHUNK = 2000
NCHUNK = EPT // CHUNK
NPAD = 200704


def _deg_body(src_h, dst_h, out, sp_dego, sp_degi, zbf, onesb, srcbuf, dstbuf,
              obuf):
    core = lax.axis_index("c")
    tid = lax.axis_index("s")

    def _fill(ref, n, val):
        def bd(i, _):
            ref[pl.ds(i * 16, 16)] = jnp.full((16,), val, ref.dtype)
            return 0
        lax.fori_loop(0, n // 16, bd, 0)

    _fill(zbf, 8192, jnp.float32(0.0))
    _fill(onesb, CHUNK, jnp.float32(1.0))

    stripe = NPAD // 16
    for sp in (sp_dego, sp_degi):
        off = 0
        while off < stripe:
            sz = min(8192, stripe - off)
            pltpu.sync_copy(zbf.at[pl.ds(0, sz)],
                            sp.at[pl.ds(tid * stripe + off, sz)])
            off += sz
    plsc.subcore_barrier()

    def chunkfn(k, _):
        base = core * (E // 2) + tid * EPT + k * CHUNK
        pltpu.sync_copy(src_h.at[pl.ds(base, CHUNK)], srcbuf)
        pltpu.sync_copy(dst_h.at[pl.ds(base, CHUNK)], dstbuf)
        pltpu.sync_copy(onesb, sp_dego.at[srcbuf], add=True)
        pltpu.sync_copy(onesb, sp_degi.at[dstbuf], add=True)
        return 0
    lax.fori_loop(0, NCHUNK, chunkfn, 0)
    plsc.subcore_barrier()

    for j, sp in enumerate((sp_dego, sp_degi)):
        off = 0
        while off < stripe:
            sz = min(8192, stripe - off)
            pltpu.sync_copy(sp.at[pl.ds(tid * stripe + off, sz)],
                            obuf.at[pl.ds(0, sz)])
            pltpu.sync_copy(
                obuf.at[pl.ds(0, sz)],
                out.at[pl.ds(core * (2 * NPAD) + j * NPAD
                             + tid * stripe + off, sz)])
            off += sz


def _degrees(src, dst):
    mesh = plsc.VectorSubcoreMesh(core_axis_name="c", subcore_axis_name="s",
                                  num_cores=NC, num_subcores=NS)
    f32 = jnp.float32
    scratch = [
        pltpu.VMEM_SHARED((NPAD,), f32),
        pltpu.VMEM_SHARED((NPAD,), f32),
        pltpu.VMEM((8192,), f32),
        pltpu.VMEM((CHUNK,), f32),
        pltpu.VMEM((CHUNK,), jnp.int32),
        pltpu.VMEM((CHUNK,), jnp.int32),
        pltpu.VMEM((8192,), f32),
    ]
    fn = functools.partial(
        pl.kernel, mesh=mesh,
        out_type=jax.ShapeDtypeStruct((NC * 2 * NPAD,), f32),
        scratch_types=scratch,
    )(_deg_body)
    p = fn(src, dst).reshape(NC, 2, NPAD)
    deg = p[0] + p[1]
    return deg[0], deg[1]


def _gcn_layer(x, src, dst, rel, rel_emb, osi, in_sqrt_inv, n_nodes,
               window=None):
    msg = x[src] * rel_emb[rel] * osi[:, None]
    if window is not None:
        msg = msg * window[:, None]
    agg = jax.ops.segment_sum(msg, dst, num_segments=n_nodes)
    return agg * in_sqrt_inv[:, None]


def _two_layer(x, src, dst, rel, rel_emb, n_nodes, window=None):
    out_deg, in_deg = _degrees(src, dst)
    out_deg = out_deg[:n_nodes]
    in_deg = in_deg[:n_nodes]
    out_sqrt_inv = jax.lax.rsqrt(jnp.maximum(out_deg, 1.0))
    in_sqrt_inv = jax.lax.rsqrt(jnp.maximum(in_deg, 1.0))
    osi = out_sqrt_inv[src]
    l1 = _gcn_layer(x, src, dst, rel, rel_emb, osi, in_sqrt_inv,
                    n_nodes, window) + x
    l2 = _gcn_layer(l1, src, dst, rel, rel_emb, osi, in_sqrt_inv,
                    n_nodes, window) + x
    return l2


def _dynamic_window(window_size, rel, window_params):
    w = jnp.clip(window_params, 0.0, 1.0)[rel, 0]
    return 1.0 / (1.0 + jnp.exp(window_size - BASE_WINDOW * w))


def _combine_body(h_ref, f_ref, g_ref, hr_ref, fr_ref, o_ref):
    o_ref[...] = (hr_ref[0] * h_ref[...] + fr_ref[0] * f_ref[...]
                  + LAMBDA_STATIC * g_ref[...])


def kernel(global_node_table, node_table, edge_table, hist_window_params,
           future_window_params, h_ratio, f_ratio, h_window, f_window,
           seed_nodes, relation_batch, glob_edge_index, glob_rel,
           h_edge_index, h_rel, f_edge_index, f_rel, neighbor_batch_size):
    i32 = jnp.int32
    gsrc = glob_edge_index[0].astype(i32)
    gdst = glob_edge_index[1].astype(i32)
    hsrc = h_edge_index[0].astype(i32)
    hdst = h_edge_index[1].astype(i32)
    fsrc = f_edge_index[0].astype(i32)
    fdst = f_edge_index[1].astype(i32)

    glob_out = _two_layer(global_node_table, gsrc, gdst, glob_rel,
                          edge_table, ENTITY_NUM, None)
    dw_h = _dynamic_window(h_window, h_rel, hist_window_params)
    h_out = _two_layer(node_table, hsrc, hdst, h_rel, edge_table,
                       N_TEMPORAL, dw_h)
    dw_f = _dynamic_window(f_window, f_rel, future_window_params)
    f_out = _two_layer(node_table, fsrc, fdst, f_rel, edge_table,
                       N_TEMPORAL, dw_f)
    orig = seed_nodes // TIME_NUM
    hs = h_out[seed_nodes]
    fs = f_out[seed_nodes]
    gs = glob_out[orig]
    out = pl.pallas_call(
        _combine_body,
        out_shape=jax.ShapeDtypeStruct(hs.shape, hs.dtype),
    )(hs, fs, gs, h_ratio, f_ratio)
    return out
